# baseline (speedup 1.0000x reference)
import functools

import jax
import jax.numpy as jnp
from jax import lax
from jax.experimental import pallas as pl
from jax.experimental.pallas import tpu as pltpu
from jax.experimental.pallas import tpu_sc as plsc

B, L, D = 4096, 200, 16
N = B * L
NW = 32
SEQ_PER_W = B // NW
C = L
NCH = SEQ_PER_W
NBUF = 4
NFULL = C // 16
TAIL = C - NFULL * 16
TV, EV = 52, 103

_mesh = plsc.VectorSubcoreMesh(core_axis_name="c", subcore_axis_name="s")


@functools.partial(
    pl.kernel,
    out_type=jax.ShapeDtypeStruct((B, L, D), jnp.float32),
    mesh=_mesh,
    compiler_params=pltpu.CompilerParams(use_tc_tiling_on_sc=False,
                                         needs_layout_passes=False),
    scratch_types=(
        [pltpu.VMEM((C + 16,), jnp.int32) for _ in range(3 * NBUF)]
        + [pltpu.VMEM((C, D), jnp.float32) for _ in range(NBUF)]
        + [pltpu.VMEM((TV * D,), jnp.float32),
           pltpu.VMEM((EV * D,), jnp.float32)]
        + [pltpu.SemaphoreType.DMA for _ in range(3 * NBUF)]
    ),
)
def _embed_sum(gid_hbm, tid_hbm, eid_hbm, gt_hbm, tt_hbm, et_hbm, out_hbm,
               *scr):
    gidx = scr[0:4]
    tidx = scr[4:8]
    eidx = scr[8:12]
    ob = scr[12:16]
    ttv, etv = scr[16], scr[17]
    isem = scr[18:22]
    gsem = scr[22:26]
    osem = scr[26:30]

    wid = lax.axis_index("s") * 2 + lax.axis_index("c")
    wseq0 = wid * SEQ_PER_W

    pltpu.sync_copy(tt_hbm, ttv)
    pltpu.sync_copy(et_hbm, etv)

    def stage_idx(ch, b):
        base = (wseq0 + ch) * C
        pltpu.async_copy(gid_hbm.at[pl.ds(base, C)],
                         gidx[b].at[pl.ds(0, C)], isem[b])
        pltpu.async_copy(tid_hbm.at[pl.ds(base, C)],
                         tidx[b].at[pl.ds(0, C)], isem[b])
        pltpu.async_copy(eid_hbm.at[pl.ds(base, C)],
                         eidx[b].at[pl.ds(0, C)], isem[b])

    def wait_idx(b):
        for ref in (gidx[b], tidx[b], eidx[b]):
            pltpu.make_async_copy(gid_hbm.at[pl.ds(0, C)],
                                  ref.at[pl.ds(0, C)], isem[b]).wait()

    def fire_gathers(b):
        pltpu.async_copy(gt_hbm.at[gidx[b].at[pl.ds(0, 128)]],
                         ob[b].at[pl.ds(0, 128)], gsem[b])
        pltpu.async_copy(gt_hbm.at[gidx[b].at[pl.ds(128, C - 128)]],
                         ob[b].at[pl.ds(128, C - 128)], gsem[b])

    def wait_gathers(b):
        pltpu.make_async_copy(gt_hbm.at[gidx[b].at[pl.ds(0, 128)]],
                              ob[b].at[pl.ds(0, 128)], gsem[b]).wait()
        pltpu.make_async_copy(gt_hbm.at[gidx[b].at[pl.ds(128, C - 128)]],
                              ob[b].at[pl.ds(128, C - 128)], gsem[b]).wait()

    def wait_out(b):
        pltpu.make_async_copy(ob[b], out_hbm.at[0], osem[b]).wait()

    lanes = lax.iota(jnp.int32, 16)
    tail_mask = lanes < TAIL

    for c in range(NBUF):
        stage_idx(c, c)
    for c in range(2):
        wait_idx(c)
        fire_gathers(c)

    def apply_small(o, ti, ei, tk, mask=None):
        tvec = ti[pl.ds(tk, 16)] * D
        evec = ei[pl.ds(tk, 16)] * D
        tokv = tk + lanes
        for d in range(D):
            tt = plsc.load_gather(ttv, [tvec + d], mask=mask)
            ee = plsc.load_gather(etv, [evec + d], mask=mask)
            plsc.addupdate_scatter(
                o, [tokv, jnp.full((16,), d, jnp.int32)], tt + ee, mask=mask)

    def ring_body(k, carry):
        for b in range(NBUF):
            ch = k * NBUF + b
            wait_gathers(b)
            o, ti, ei = ob[b], tidx[b], eidx[b]

            @plsc.parallel_loop(0, NFULL, unroll=2)
            def _(grp):
                apply_small(o, ti, ei, grp * 16)

            apply_small(o, ti, ei, NFULL * 16, mask=tail_mask)

            pltpu.async_copy(o, out_hbm.at[wseq0 + ch], osem[b])

            @pl.when(ch + NBUF < NCH)
            def _():
                stage_idx(ch + NBUF, b)

            bg = (b + 2) % NBUF

            @pl.when(ch + 2 < NCH)
            def _():
                @pl.when(ch >= 2)
                def _():
                    wait_out(bg)
                wait_idx(bg)
                fire_gathers(bg)
        return carry

    lax.fori_loop(0, NCH // NBUF, ring_body, 0)
    for b in range(NBUF):
        wait_out(b)


def kernel(grid, timestamp, event, train_mode, grid_table, time_table, event_table):
    gid = grid.reshape(N).astype(jnp.int32)
    tid = timestamp.reshape(N).astype(jnp.int32)
    eid = event.reshape(N).astype(jnp.int32)
    return _embed_sum(gid, tid, eid, grid_table,
                      time_table.reshape(TV * D), event_table.reshape(EV * D))


# speedup vs baseline: 2.3575x; 2.3575x over previous
import functools

import jax
import jax.numpy as jnp
from jax import lax
from jax.experimental import pallas as pl
from jax.experimental.pallas import tpu as pltpu
from jax.experimental.pallas import tpu_sc as plsc

B, L, D = 4096, 200, 16
NW = 32
BT = B // NW
LB = 5
NCH = L // LB
NBUF = 4
TV, EV = 52, 103

_mesh = plsc.VectorSubcoreMesh(core_axis_name="c", subcore_axis_name="s")

V = 1000004
VFULL = (V // 128) * 128
NBLK = VFULL // 128
VTAIL = V - VFULL


@functools.partial(
    pl.kernel,
    out_type=jax.ShapeDtypeStruct((V * D,), jnp.float32),
    mesh=_mesh,
    compiler_params=pltpu.CompilerParams(use_tc_tiling_on_sc=True,
                                         needs_layout_passes=False),
    scratch_types=(
        [pltpu.VMEM((D, 128), jnp.float32) for _ in range(2)]
        + [pltpu.VMEM((128 * D,), jnp.float32) for _ in range(2)]
        + [pltpu.SemaphoreType.DMA for _ in range(4)]
    ),
)
def _detile_table(gtt_hbm, tail_hbm, out_hbm, *scr):
    blk = scr[0:2]
    tbuf = scr[2:4]
    gsem = scr[4:6]
    osem = scr[6:8]

    wid = lax.axis_index("s") * 2 + lax.axis_index("c")
    lanes = lax.iota(jnp.int32, 16)
    kmax = NBLK // NW + 1

    def fire_in(k, b):
        c0 = (k * NW + wid) * 128
        pltpu.async_copy(gtt_hbm.at[:, pl.ds(c0, 128)], blk[b], gsem[b])

    def wait_in(b):
        pltpu.make_async_copy(gtt_hbm.at[:, pl.ds(0, 128)], blk[b],
                              gsem[b]).wait()

    def wait_out(b):
        pltpu.make_async_copy(tbuf[b], out_hbm.at[pl.ds(0, 128 * D)],
                              osem[b]).wait()

    @pl.when(wid == 0)
    def _():
        pltpu.sync_copy(tail_hbm, tbuf[0].at[pl.ds(0, VTAIL * D)])
        pltpu.sync_copy(tbuf[0].at[pl.ds(0, VTAIL * D)],
                        out_hbm.at[pl.ds(VFULL * D, VTAIL * D)])

    fire_in(0, 0)

    def body(k, carry):
        for b in range(2):
            kk = 2 * k + b
            cblk = kk * NW + wid

            @pl.when(cblk < NBLK)
            def _():
                wait_in(b)

                @pl.when((kk + 1) * NW + wid < NBLK)
                def _():
                    fire_in(kk + 1, (b + 1) % 2)

                @pl.when(kk >= 2)
                def _():
                    wait_out(b)

                @plsc.parallel_loop(0, 128, unroll=4)
                def _(c):
                    tbuf[b][pl.ds(c * D, 16)] = plsc.load_gather(
                        blk[b], [lanes, jnp.full((16,), 0, jnp.int32) + c])

                pltpu.async_copy(tbuf[b], out_hbm.at[pl.ds(cblk * 128 * D,
                                                           128 * D)], osem[b])
        return carry

    lax.fori_loop(0, (kmax + 1) // 2, body, 0)
    for b in range(2):
        wait_out(b)


@functools.partial(
    pl.kernel,
    out_type=jax.ShapeDtypeStruct((L, 2, NW, 8 * BT), jnp.float32),
    mesh=_mesh,
    compiler_params=pltpu.CompilerParams(use_tc_tiling_on_sc=False,
                                         needs_layout_passes=False),
    scratch_types=(
        [pltpu.VMEM((LB, BT), jnp.int32) for _ in range(3 * NBUF)]
        + [pltpu.VMEM((LB, BT, D), jnp.float32) for _ in range(NBUF)]
        + [pltpu.VMEM((LB * 2 * 8 * BT,), jnp.float32) for _ in range(NBUF)]
        + [pltpu.VMEM((TV * D,), jnp.float32),
           pltpu.VMEM((EV * D,), jnp.float32)]
        + [pltpu.SemaphoreType.DMA for _ in range(3 * NBUF)]
    ),
)
def _embed_sum(gid_hbm, tid_hbm, eid_hbm, gt_hbm, tt_hbm, et_hbm, out_hbm,
               *scr):
    gidx = scr[0:4]
    tidx = scr[4:8]
    eidx = scr[8:12]
    rb = scr[12:16]
    pb = scr[16:20]
    ttv, etv = scr[20], scr[21]
    isem = scr[22:26]
    gsem = scr[26:30]
    osem = scr[30:34]

    wid = lax.axis_index("s") * 2 + lax.axis_index("c")
    wb0 = wid * BT

    pltpu.sync_copy(tt_hbm, ttv)
    pltpu.sync_copy(et_hbm, etv)

    def stage_idx(ch, b):
        l0 = ch * LB
        for ih, dst in ((gid_hbm, gidx[b]), (tid_hbm, tidx[b]),
                        (eid_hbm, eidx[b])):
            pltpu.async_copy(ih.at[pl.ds(l0, LB), pl.ds(wb0, BT)], dst,
                             isem[b])

    def wait_idx(b):
        for dst in (gidx[b], tidx[b], eidx[b]):
            pltpu.make_async_copy(gid_hbm.at[pl.ds(0, LB), pl.ds(0, BT)],
                                  dst, isem[b]).wait()

    def fire_gathers(b):
        for li in range(LB):
            pltpu.async_copy(gt_hbm.at[gidx[b].at[li]], rb[b].at[li], gsem[b])

    def wait_gathers(b):
        for li in range(LB):
            pltpu.make_async_copy(gt_hbm.at[gidx[b].at[li]], rb[b].at[li],
                                  gsem[b]).wait()

    def fire_out(ch, b):
        l0 = ch * LB
        for li in range(LB):
            for dh in range(2):
                pltpu.async_copy(
                    pb[b].at[pl.ds((li * 2 + dh) * 8 * BT, 8 * BT)],
                    out_hbm.at[l0 + li, dh, wid], osem[b])

    def wait_out(b):
        for _ in range(2 * LB):
            pltpu.make_async_copy(pb[b].at[pl.ds(0, 8 * BT)],
                                  out_hbm.at[0, 0, 0], osem[b]).wait()

    lanes = lax.iota(jnp.int32, 16)

    for c in range(NBUF):
        stage_idx(c, c)
    for c in range(2):
        wait_idx(c)
        fire_gathers(c)

    def ring_body(k, carry):
        for b in range(NBUF):
            ch = k * NBUF + b
            wait_gathers(b)

            @pl.when(ch >= NBUF)
            def _():
                wait_out(b)

            r, ti, ei, o = rb[b], tidx[b], eidx[b], pb[b]

            @plsc.parallel_loop(0, LB * (BT // 16))
            def _(g):
                li = g // (BT // 16)
                tk = (g % (BT // 16)) * 16
                tokv = tk + lanes
                tvec = ti[li, pl.ds(tk, 16)] * D
                evec = ei[li, pl.ds(tk, 16)] * D
                lisp = jnp.full((16,), 0, jnp.int32) + li
                base = li * (2 * 8 * BT) + tk
                for d in range(D):
                    col = (plsc.load_gather(r, [lisp, tokv, jnp.full((16,), d, jnp.int32)])
                           + plsc.load_gather(ttv, [tvec + d])
                           + plsc.load_gather(etv, [evec + d]))
                    o[pl.ds(base + d * BT, 16)] = col

            fire_out(ch, b)

            @pl.when(ch + NBUF < NCH)
            def _():
                stage_idx(ch + NBUF, b)

            bg = (b + 2) % NBUF

            @pl.when(ch + 2 < NCH)
            def _():
                wait_idx(bg)
                fire_gathers(bg)
        return carry

    lax.fori_loop(0, NCH // NBUF, ring_body, 0)
    for b in range(NBUF):
        wait_out(b)


def kernel(grid, timestamp, event, train_mode, grid_table, time_table, event_table):
    gid = grid.T.astype(jnp.int32)
    tid = timestamp.T.astype(jnp.int32)
    eid = event.T.astype(jnp.int32)
    gt_lin = _detile_table(grid_table.T,
                           grid_table[VFULL:].reshape(VTAIL * D))
    gt_rows = gt_lin.reshape(V, D)
    out_p = _embed_sum(gid, tid, eid, gt_rows,
                       time_table.reshape(TV * D), event_table.reshape(EV * D))
    out = (out_p.reshape(L, 2, NW, 8, BT)
           .transpose(2, 4, 0, 1, 3)
           .reshape(B, L, D))
    return out


# speedup vs baseline: 2.3599x; 1.0010x over previous
import functools

import jax
import jax.numpy as jnp
from jax import lax
from jax.experimental import pallas as pl
from jax.experimental.pallas import tpu as pltpu
from jax.experimental.pallas import tpu_sc as plsc

B, L, D = 4096, 200, 16
NW = 32
BT = B // NW
LB = 5
NCH = L // LB
NBUF = 4
TV, EV = 52, 103

_mesh = plsc.VectorSubcoreMesh(core_axis_name="c", subcore_axis_name="s")

V = 1000004
VFULL = (V // 128) * 128
NBLK = VFULL // 128
VTAIL = V - VFULL


@functools.partial(
    pl.kernel,
    out_type=jax.ShapeDtypeStruct((V * D,), jnp.float32),
    mesh=_mesh,
    compiler_params=pltpu.CompilerParams(use_tc_tiling_on_sc=True,
                                         needs_layout_passes=False),
    scratch_types=(
        [pltpu.VMEM((D, 128), jnp.float32) for _ in range(4)]
        + [pltpu.VMEM((128 * D,), jnp.float32) for _ in range(4)]
        + [pltpu.SemaphoreType.DMA for _ in range(8)]
    ),
)
def _detile_table(gtt_hbm, tail_hbm, out_hbm, *scr):
    blk = scr[0:4]
    tbuf = scr[4:8]
    gsem = scr[8:12]
    osem = scr[12:16]

    wid = lax.axis_index("s") * 2 + lax.axis_index("c")
    lanes = lax.iota(jnp.int32, 16)
    kmax = NBLK // NW + 1

    def fire_in(k, b):
        c0 = (k * NW + wid) * 128
        pltpu.async_copy(gtt_hbm.at[:, pl.ds(c0, 128)], blk[b], gsem[b])

    def wait_in(b):
        pltpu.make_async_copy(gtt_hbm.at[:, pl.ds(0, 128)], blk[b],
                              gsem[b]).wait()

    def wait_out(b):
        pltpu.make_async_copy(tbuf[b], out_hbm.at[pl.ds(0, 128 * D)],
                              osem[b]).wait()

    @pl.when(wid == 0)
    def _():
        pltpu.sync_copy(tail_hbm, tbuf[0].at[pl.ds(0, VTAIL * D)])
        pltpu.sync_copy(tbuf[0].at[pl.ds(0, VTAIL * D)],
                        out_hbm.at[pl.ds(VFULL * D, VTAIL * D)])

    for kk in range(3):
        fire_in(kk, kk)

    def body(k, carry):
        for b in range(4):
            kk = 4 * k + b
            cblk = kk * NW + wid

            @pl.when(cblk < NBLK)
            def _():
                wait_in(b)

                @pl.when((kk + 3) * NW + wid < NBLK)
                def _():
                    fire_in(kk + 3, (b + 3) % 4)

                @pl.when(kk >= 4)
                def _():
                    wait_out(b)

                @plsc.parallel_loop(0, 128, unroll=4)
                def _(c):
                    tbuf[b][pl.ds(c * D, 16)] = plsc.load_gather(
                        blk[b], [lanes, jnp.full((16,), 0, jnp.int32) + c])

                pltpu.async_copy(tbuf[b], out_hbm.at[pl.ds(cblk * 128 * D,
                                                           128 * D)], osem[b])
        return carry

    lax.fori_loop(0, (kmax + 3) // 4, body, 0)
    for b in range(4):
        wait_out(b)


@functools.partial(
    pl.kernel,
    out_type=jax.ShapeDtypeStruct((L, 2, NW, 8 * BT), jnp.float32),
    mesh=_mesh,
    compiler_params=pltpu.CompilerParams(use_tc_tiling_on_sc=False,
                                         needs_layout_passes=False),
    scratch_types=(
        [pltpu.VMEM((LB, BT), jnp.int32) for _ in range(3 * NBUF)]
        + [pltpu.VMEM((LB, BT, D), jnp.float32) for _ in range(NBUF)]
        + [pltpu.VMEM((LB * 2 * 8 * BT,), jnp.float32) for _ in range(NBUF)]
        + [pltpu.VMEM((TV * D,), jnp.float32),
           pltpu.VMEM((EV * D,), jnp.float32)]
        + [pltpu.SemaphoreType.DMA for _ in range(3 * NBUF)]
    ),
)
def _embed_sum(gid_hbm, tid_hbm, eid_hbm, gt_hbm, tt_hbm, et_hbm, out_hbm,
               *scr):
    gidx = scr[0:4]
    tidx = scr[4:8]
    eidx = scr[8:12]
    rb = scr[12:16]
    pb = scr[16:20]
    ttv, etv = scr[20], scr[21]
    isem = scr[22:26]
    gsem = scr[26:30]
    osem = scr[30:34]

    wid = lax.axis_index("s") * 2 + lax.axis_index("c")
    wb0 = wid * BT

    pltpu.sync_copy(tt_hbm, ttv)
    pltpu.sync_copy(et_hbm, etv)

    def stage_idx(ch, b):
        l0 = ch * LB
        for ih, dst in ((gid_hbm, gidx[b]), (tid_hbm, tidx[b]),
                        (eid_hbm, eidx[b])):
            pltpu.async_copy(ih.at[pl.ds(l0, LB), pl.ds(wb0, BT)], dst,
                             isem[b])

    def wait_idx(b):
        for dst in (gidx[b], tidx[b], eidx[b]):
            pltpu.make_async_copy(gid_hbm.at[pl.ds(0, LB), pl.ds(0, BT)],
                                  dst, isem[b]).wait()

    def fire_gathers(b):
        for li in range(LB):
            pltpu.async_copy(gt_hbm.at[gidx[b].at[li]], rb[b].at[li], gsem[b])

    def wait_gathers(b):
        for li in range(LB):
            pltpu.make_async_copy(gt_hbm.at[gidx[b].at[li]], rb[b].at[li],
                                  gsem[b]).wait()

    def fire_out(ch, b):
        l0 = ch * LB
        for li in range(LB):
            for dh in range(2):
                pltpu.async_copy(
                    pb[b].at[pl.ds((li * 2 + dh) * 8 * BT, 8 * BT)],
                    out_hbm.at[l0 + li, dh, wid], osem[b])

    def wait_out(b):
        for _ in range(2 * LB):
            pltpu.make_async_copy(pb[b].at[pl.ds(0, 8 * BT)],
                                  out_hbm.at[0, 0, 0], osem[b]).wait()

    lanes = lax.iota(jnp.int32, 16)

    for c in range(NBUF):
        stage_idx(c, c)
    for c in range(3):
        wait_idx(c)
        fire_gathers(c)

    def ring_body(k, carry):
        for b in range(NBUF):
            ch = k * NBUF + b
            wait_gathers(b)

            @pl.when(ch >= NBUF)
            def _():
                wait_out(b)

            r, ti, ei, o = rb[b], tidx[b], eidx[b], pb[b]

            @plsc.parallel_loop(0, LB * (BT // 16))
            def _(g):
                li = g // (BT // 16)
                tk = (g % (BT // 16)) * 16
                tokv = tk + lanes
                tvec = ti[li, pl.ds(tk, 16)] * D
                evec = ei[li, pl.ds(tk, 16)] * D
                lisp = jnp.full((16,), 0, jnp.int32) + li
                base = li * (2 * 8 * BT) + tk
                for d in range(D):
                    col = (plsc.load_gather(r, [lisp, tokv, jnp.full((16,), d, jnp.int32)])
                           + plsc.load_gather(ttv, [tvec + d])
                           + plsc.load_gather(etv, [evec + d]))
                    o[pl.ds(base + d * BT, 16)] = col

            fire_out(ch, b)

            @pl.when(ch + NBUF < NCH)
            def _():
                stage_idx(ch + NBUF, b)

            bg = (b + 3) % NBUF

            @pl.when(ch + 3 < NCH)
            def _():
                wait_idx(bg)
                fire_gathers(bg)
        return carry

    lax.fori_loop(0, NCH // NBUF, ring_body, 0)
    for b in range(NBUF):
        wait_out(b)


def kernel(grid, timestamp, event, train_mode, grid_table, time_table, event_table):
    gid = grid.T.astype(jnp.int32)
    tid = timestamp.T.astype(jnp.int32)
    eid = event.T.astype(jnp.int32)
    gt_lin = _detile_table(grid_table.T,
                           grid_table[VFULL:].reshape(VTAIL * D))
    gt_rows = gt_lin.reshape(V, D)
    out_p = _embed_sum(gid, tid, eid, gt_rows,
                       time_table.reshape(TV * D), event_table.reshape(EV * D))
    out = (out_p.reshape(L, 2, NW, 8, BT)
           .transpose(2, 4, 0, 1, 3)
           .reshape(B, L, D))
    return out


# speedup vs baseline: 3.5976x; 1.5245x over previous
import functools

import jax
import jax.numpy as jnp
from jax import lax
from jax.experimental import pallas as pl
from jax.experimental.pallas import tpu as pltpu
from jax.experimental.pallas import tpu_sc as plsc

B, L, D = 4096, 200, 16
NW = 32
BT = B // NW
LB = 5
NCH = L // LB
NBUF = 4
TV, EV = 52, 103

_mesh = plsc.VectorSubcoreMesh(core_axis_name="c", subcore_axis_name="s")

V = 1000004
W = 128
VFULL = (V // W) * W
NBLK = VFULL // W
VTAIL = V - VFULL


@functools.partial(
    pl.kernel,
    out_type=jax.ShapeDtypeStruct((V * D,), jnp.float32),
    mesh=_mesh,
    compiler_params=pltpu.CompilerParams(use_tc_tiling_on_sc=True,
                                         needs_layout_passes=False),
    scratch_types=(
        [pltpu.VMEM((D, W + 1), jnp.float32) for _ in range(4)]
        + [pltpu.VMEM((W * D,), jnp.float32) for _ in range(4)]
        + [pltpu.SemaphoreType.DMA for _ in range(8)]
    ),
)
def _detile_table(gtt_hbm, tail_hbm, out_hbm, *scr):
    blk = scr[0:4]
    tbuf = scr[4:8]
    gsem = scr[8:12]
    osem = scr[12:16]

    wid = lax.axis_index("s") * 2 + lax.axis_index("c")
    lanes = lax.iota(jnp.int32, 16)
    kmax = NBLK // NW + 1

    def fire_in(k, b):
        c0 = (k * NW + wid) * W
        pltpu.async_copy(gtt_hbm.at[:, pl.ds(c0, W)],
                         blk[b].at[:, pl.ds(0, W)], gsem[b])

    def wait_in(b):
        pltpu.make_async_copy(gtt_hbm.at[:, pl.ds(0, W)],
                              blk[b].at[:, pl.ds(0, W)], gsem[b]).wait()

    def wait_out(b):
        pltpu.make_async_copy(tbuf[b], out_hbm.at[pl.ds(0, W * D)],
                              osem[b]).wait()

    @pl.when(wid == 0)
    def _():
        pltpu.sync_copy(tail_hbm, tbuf[0].at[pl.ds(0, VTAIL * D)])
        pltpu.sync_copy(tbuf[0].at[pl.ds(0, VTAIL * D)],
                        out_hbm.at[pl.ds(VFULL * D, VTAIL * D)])

    for kk in range(3):
        fire_in(kk, kk)

    def body(k, carry):
        for b in range(4):
            kk = 4 * k + b
            cblk = kk * NW + wid

            @pl.when(cblk < NBLK)
            def _():
                wait_in(b)

                @pl.when((kk + 3) * NW + wid < NBLK)
                def _():
                    fire_in(kk + 3, (b + 3) % 4)

                @pl.when(kk >= 4)
                def _():
                    wait_out(b)

                @plsc.parallel_loop(0, W, unroll=4)
                def _(c):
                    tbuf[b][pl.ds(c * D, 16)] = plsc.load_gather(
                        blk[b], [lanes, jnp.full((16,), 0, jnp.int32) + c])

                pltpu.async_copy(tbuf[b], out_hbm.at[pl.ds(cblk * W * D,
                                                           W * D)], osem[b])
        return carry

    lax.fori_loop(0, (kmax + 3) // 4, body, 0)
    for b in range(4):
        wait_out(b)


@functools.partial(
    pl.kernel,
    out_type=jax.ShapeDtypeStruct((L, 2, NW, 8 * BT), jnp.float32),
    mesh=_mesh,
    compiler_params=pltpu.CompilerParams(use_tc_tiling_on_sc=False,
                                         needs_layout_passes=False),
    scratch_types=(
        [pltpu.VMEM((LB, BT), jnp.int32) for _ in range(3 * NBUF)]
        + [pltpu.VMEM((LB, BT, D), jnp.float32) for _ in range(NBUF)]
        + [pltpu.VMEM((LB * 2 * 8 * BT,), jnp.float32) for _ in range(NBUF)]
        + [pltpu.VMEM((TV * (D + 1),), jnp.float32),
           pltpu.VMEM((EV * (D + 1),), jnp.float32)]
        + [pltpu.SemaphoreType.DMA for _ in range(3 * NBUF)]
    ),
)
def _embed_sum(gid_hbm, tid_hbm, eid_hbm, gt_hbm, tt_hbm, et_hbm, out_hbm,
               *scr):
    gidx = scr[0:4]
    tidx = scr[4:8]
    eidx = scr[8:12]
    rb = scr[12:16]
    pb = scr[16:20]
    ttv, etv = scr[20], scr[21]
    isem = scr[22:26]
    gsem = scr[26:30]
    osem = scr[30:34]

    wid = lax.axis_index("s") * 2 + lax.axis_index("c")
    wb0 = wid * BT

    pltpu.sync_copy(tt_hbm, ttv)
    pltpu.sync_copy(et_hbm, etv)

    def stage_idx(ch, b):
        l0 = ch * LB
        for ih, dst in ((gid_hbm, gidx[b]), (tid_hbm, tidx[b]),
                        (eid_hbm, eidx[b])):
            pltpu.async_copy(ih.at[pl.ds(l0, LB), pl.ds(wb0, BT)], dst,
                             isem[b])

    def wait_idx(b):
        for dst in (gidx[b], tidx[b], eidx[b]):
            pltpu.make_async_copy(gid_hbm.at[pl.ds(0, LB), pl.ds(0, BT)],
                                  dst, isem[b]).wait()

    def fire_gathers(b):
        for li in range(LB):
            pltpu.async_copy(gt_hbm.at[gidx[b].at[li]], rb[b].at[li], gsem[b])

    def wait_gathers(b):
        for li in range(LB):
            pltpu.make_async_copy(gt_hbm.at[gidx[b].at[li]], rb[b].at[li],
                                  gsem[b]).wait()

    def fire_out(ch, b):
        l0 = ch * LB
        for li in range(LB):
            for dh in range(2):
                pltpu.async_copy(
                    pb[b].at[pl.ds((li * 2 + dh) * 8 * BT, 8 * BT)],
                    out_hbm.at[l0 + li, dh, wid], osem[b])

    def wait_out(b):
        for _ in range(2 * LB):
            pltpu.make_async_copy(pb[b].at[pl.ds(0, 8 * BT)],
                                  out_hbm.at[0, 0, 0], osem[b]).wait()

    lanes = lax.iota(jnp.int32, 16)

    for c in range(NBUF):
        stage_idx(c, c)
    for c in range(3):
        wait_idx(c)
        fire_gathers(c)

    def ring_body(k, carry):
        for b in range(NBUF):
            ch = k * NBUF + b
            wait_gathers(b)

            @pl.when(ch >= NBUF)
            def _():
                wait_out(b)

            r, ti, ei, o = rb[b], tidx[b], eidx[b], pb[b]

            @plsc.parallel_loop(0, LB * (BT // 16))
            def _(g):
                li = g // (BT // 16)
                tk = (g % (BT // 16)) * 16
                tokv = tk + lanes
                tvec = ti[li, pl.ds(tk, 16)] * (D + 1)
                evec = ei[li, pl.ds(tk, 16)] * (D + 1)
                lisp = jnp.full((16,), 0, jnp.int32) + li
                base = li * (2 * 8 * BT) + tk
                for d in range(D):
                    col = (plsc.load_gather(r, [lisp, tokv, jnp.full((16,), d, jnp.int32)])
                           + plsc.load_gather(ttv, [tvec + d])
                           + plsc.load_gather(etv, [evec + d]))
                    o[pl.ds(base + d * BT, 16)] = col

            fire_out(ch, b)

            @pl.when(ch + NBUF < NCH)
            def _():
                stage_idx(ch + NBUF, b)

            bg = (b + 3) % NBUF

            @pl.when(ch + 3 < NCH)
            def _():
                wait_idx(bg)
                fire_gathers(bg)
        return carry

    lax.fori_loop(0, NCH // NBUF, ring_body, 0)
    for b in range(NBUF):
        wait_out(b)


def kernel(grid, timestamp, event, train_mode, grid_table, time_table, event_table):
    gid = grid.T.astype(jnp.int32)
    tid = timestamp.T.astype(jnp.int32)
    eid = event.T.astype(jnp.int32)
    gt_lin = _detile_table(grid_table.T,
                           grid_table[VFULL:].reshape(VTAIL * D))
    gt_rows = gt_lin.reshape(V, D)
    ttp = jnp.pad(time_table, ((0, 0), (0, 1))).reshape(TV * (D + 1))
    etp = jnp.pad(event_table, ((0, 0), (0, 1))).reshape(EV * (D + 1))
    out_p = _embed_sum(gid, tid, eid, gt_rows, ttp, etp)
    out = (out_p.reshape(L, 2, NW, 8, BT)
           .transpose(2, 4, 0, 1, 3)
           .reshape(B, L, D))
    return out


# speedup vs baseline: 3.6779x; 1.0223x over previous
import functools

import jax
import jax.numpy as jnp
from jax import lax
from jax.experimental import pallas as pl
from jax.experimental.pallas import tpu as pltpu
from jax.experimental.pallas import tpu_sc as plsc

B, L, D = 4096, 200, 16
NW = 32
BT = B // NW
LB = 5
NCH = L // LB
NBUF = 4
TV, EV = 52, 103

_mesh = plsc.VectorSubcoreMesh(core_axis_name="c", subcore_axis_name="s")

V = 1000004
W = 128
VFULL = (V // W) * W
NBLK = VFULL // W
VTAIL = V - VFULL


@functools.partial(
    pl.kernel,
    out_type=jax.ShapeDtypeStruct((V * D,), jnp.float32),
    mesh=_mesh,
    compiler_params=pltpu.CompilerParams(use_tc_tiling_on_sc=True,
                                         needs_layout_passes=False),
    scratch_types=(
        [pltpu.VMEM((D, W + 1), jnp.float32) for _ in range(4)]
        + [pltpu.VMEM((W * D,), jnp.float32) for _ in range(4)]
        + [pltpu.SemaphoreType.DMA for _ in range(8)]
    ),
)
def _detile_table(gtt_hbm, tail_hbm, out_hbm, *scr):
    blk = scr[0:4]
    tbuf = scr[4:8]
    gsem = scr[8:12]
    osem = scr[12:16]

    wid = lax.axis_index("s") * 2 + lax.axis_index("c")
    lanes = lax.iota(jnp.int32, 16)
    kmax = NBLK // NW + 1

    def fire_in(k, b):
        c0 = (k * NW + wid) * W
        pltpu.async_copy(gtt_hbm.at[:, pl.ds(c0, W)],
                         blk[b].at[:, pl.ds(0, W)], gsem[b])

    def wait_in(b):
        pltpu.make_async_copy(gtt_hbm.at[:, pl.ds(0, W)],
                              blk[b].at[:, pl.ds(0, W)], gsem[b]).wait()

    def wait_out(b):
        pltpu.make_async_copy(tbuf[b], out_hbm.at[pl.ds(0, W * D)],
                              osem[b]).wait()

    @pl.when(wid == 0)
    def _():
        pltpu.sync_copy(tail_hbm, tbuf[0].at[pl.ds(0, VTAIL * D)])
        pltpu.sync_copy(tbuf[0].at[pl.ds(0, VTAIL * D)],
                        out_hbm.at[pl.ds(VFULL * D, VTAIL * D)])

    for kk in range(3):
        fire_in(kk, kk)

    def body(k, carry):
        for b in range(4):
            kk = 4 * k + b
            cblk = kk * NW + wid

            @pl.when(cblk < NBLK)
            def _():
                wait_in(b)

                @pl.when((kk + 3) * NW + wid < NBLK)
                def _():
                    fire_in(kk + 3, (b + 3) % 4)

                @pl.when(kk >= 4)
                def _():
                    wait_out(b)

                @plsc.parallel_loop(0, W, unroll=4)
                def _(c):
                    tbuf[b][pl.ds(c * D, 16)] = plsc.load_gather(
                        blk[b], [lanes, jnp.full((16,), 0, jnp.int32) + c])

                pltpu.async_copy(tbuf[b], out_hbm.at[pl.ds(cblk * W * D,
                                                           W * D)], osem[b])
        return carry

    lax.fori_loop(0, (kmax + 3) // 4, body, 0)
    for b in range(4):
        wait_out(b)


@functools.partial(
    pl.kernel,
    out_type=jax.ShapeDtypeStruct((L, 2, NW, 8, BT), jnp.float32),
    mesh=_mesh,
    compiler_params=pltpu.CompilerParams(use_tc_tiling_on_sc=False,
                                         needs_layout_passes=False),
    scratch_types=(
        [pltpu.VMEM((LB, BT), jnp.int32) for _ in range(3 * NBUF)]
        + [pltpu.VMEM((LB, BT, D), jnp.float32) for _ in range(NBUF)]
        + [pltpu.VMEM((LB * 2 * 8, BT + 1), jnp.float32) for _ in range(NBUF)]
        + [pltpu.VMEM((TV * (D + 1),), jnp.float32),
           pltpu.VMEM((EV * (D + 1),), jnp.float32)]
        + [pltpu.SemaphoreType.DMA for _ in range(3 * NBUF)]
    ),
)
def _embed_sum(gid_hbm, tid_hbm, eid_hbm, gt_hbm, tt_hbm, et_hbm, out_hbm,
               *scr):
    gidx = scr[0:4]
    tidx = scr[4:8]
    eidx = scr[8:12]
    rb = scr[12:16]
    pb = scr[16:20]
    ttv, etv = scr[20], scr[21]
    isem = scr[22:26]
    gsem = scr[26:30]
    osem = scr[30:34]

    wid = lax.axis_index("s") * 2 + lax.axis_index("c")
    wb0 = wid * BT

    pltpu.sync_copy(tt_hbm, ttv)
    pltpu.sync_copy(et_hbm, etv)

    def stage_idx(ch, b):
        l0 = ch * LB
        for ih, dst in ((gid_hbm, gidx[b]), (tid_hbm, tidx[b]),
                        (eid_hbm, eidx[b])):
            pltpu.async_copy(ih.at[pl.ds(l0, LB), pl.ds(wb0, BT)], dst,
                             isem[b])

    def wait_idx(b):
        for dst in (gidx[b], tidx[b], eidx[b]):
            pltpu.make_async_copy(gid_hbm.at[pl.ds(0, LB), pl.ds(0, BT)],
                                  dst, isem[b]).wait()

    def fire_gathers(b):
        for li in range(LB):
            pltpu.async_copy(gt_hbm.at[gidx[b].at[li]], rb[b].at[li], gsem[b])

    def wait_gathers(b):
        for li in range(LB):
            pltpu.make_async_copy(gt_hbm.at[gidx[b].at[li]], rb[b].at[li],
                                  gsem[b]).wait()

    def fire_out(ch, b):
        l0 = ch * LB
        for li in range(LB):
            for dh in range(2):
                pltpu.async_copy(
                    pb[b].at[pl.ds((li * 2 + dh) * 8, 8), pl.ds(0, BT)],
                    out_hbm.at[l0 + li, dh, wid], osem[b])

    def wait_out(b):
        for _ in range(2 * LB):
            pltpu.make_async_copy(pb[b].at[pl.ds(0, 8), pl.ds(0, BT)],
                                  out_hbm.at[0, 0, 0], osem[b]).wait()

    lanes = lax.iota(jnp.int32, 16)

    for c in range(NBUF):
        stage_idx(c, c)
    for c in range(3):
        wait_idx(c)
        fire_gathers(c)

    def ring_body(k, carry):
        for b in range(NBUF):
            ch = k * NBUF + b
            wait_gathers(b)

            @pl.when(ch >= NBUF)
            def _():
                wait_out(b)

            r, ti, ei, o = rb[b], tidx[b], eidx[b], pb[b]

            @plsc.parallel_loop(0, LB * (BT // 16))
            def _(g):
                li = g // (BT // 16)
                tk = (g % (BT // 16)) * 16
                tvec = ti[li, pl.ds(tk, 16)] * (D + 1)
                evec = ei[li, pl.ds(tk, 16)] * (D + 1)
                rowb = li * D
                for j in range(16):
                    plsc.store_scatter(
                        o, [rowb + lanes, jnp.full((16,), 0, jnp.int32) + (tk + j)],
                        r[li, tk + j])
                for d in range(D):
                    col = (plsc.load_gather(ttv, [tvec + d])
                           + plsc.load_gather(etv, [evec + d]))
                    plsc.addupdate(o.at[rowb + d, pl.ds(tk, 16)], col)

            fire_out(ch, b)

            @pl.when(ch + NBUF < NCH)
            def _():
                stage_idx(ch + NBUF, b)

            bg = (b + 3) % NBUF

            @pl.when(ch + 3 < NCH)
            def _():
                wait_idx(bg)
                fire_gathers(bg)
        return carry

    lax.fori_loop(0, NCH // NBUF, ring_body, 0)
    for b in range(NBUF):
        wait_out(b)


def kernel(grid, timestamp, event, train_mode, grid_table, time_table, event_table):
    gid = grid.T.astype(jnp.int32)
    tid = timestamp.T.astype(jnp.int32)
    eid = event.T.astype(jnp.int32)
    gt_lin = _detile_table(grid_table.T,
                           grid_table[VFULL:].reshape(VTAIL * D))
    gt_rows = gt_lin.reshape(V, D)
    ttp = jnp.pad(time_table, ((0, 0), (0, 1))).reshape(TV * (D + 1))
    etp = jnp.pad(event_table, ((0, 0), (0, 1))).reshape(EV * (D + 1))
    out_p = _embed_sum(gid, tid, eid, gt_rows, ttp, etp)
    out = (out_p.transpose(2, 4, 0, 1, 3)
           .reshape(B, L, D))
    return out
